# baseline (speedup 1.0000x reference)
import functools

import jax
import jax.numpy as jnp
from jax import lax
from jax.experimental import pallas as pl
from jax.experimental.pallas import tpu as pltpu
from jax.experimental.pallas import tpu_sc as plsc

POSK = 32
TOPK = POSK + 1
CHUNK = 128
NSEL = 36
RHEADS = 8
TILE = 1024
GCHUNK = 128
NEG = -1e30
IMAX = 2**31 - 1


def _sim_body(nvalid, tile, chunk, vq_ref, aq_ref, vt_ref, at_ref,
              sim_ref, cmax_ref):
    t = pl.program_id(0)

    def norm_rows(x):
        ss = jnp.sum(x * x, axis=1, keepdims=True)
        return x / jnp.sqrt(jnp.maximum(ss, 1e-30))

    vqn = norm_rows(vq_ref[...])
    aqn = norm_rows(aq_ref[...])
    vtn = norm_rows(vt_ref[...])
    atn = norm_rows(at_ref[...])
    dn = (((1,), (1,)), ((), ()))
    sv = lax.dot_general(vqn, vtn, dn, preferred_element_type=jnp.float32)
    sa = lax.dot_general(aqn, atn, dn, preferred_element_type=jnp.float32)
    s = jnp.minimum(sv, sa)
    nidx = t * tile + lax.broadcasted_iota(jnp.int32, (1, tile), 1)
    s = jnp.where(nidx < nvalid, s, NEG)
    q = s.shape[0]
    s3 = s.reshape(q, tile // chunk, chunk)
    sim_ref[...] = s3
    cmax_ref[...] = jnp.max(s3, axis=2)[None]


def _chunksel_body(nsel, nchunks, cmax_ref, flat_ref):
    m = cmax_ref[...]
    q = m.shape[0]
    ciota = lax.broadcasted_iota(jnp.int32, (q, nchunks), 1)
    qiota = lax.broadcasted_iota(jnp.int32, (q, 1), 0)
    cols = []
    for _ in range(nsel):
        mx = jnp.max(m, axis=1, keepdims=True)
        sel = jnp.min(jnp.where(m == mx, ciota, IMAX), axis=1, keepdims=True)
        cols.append(sel)
        m = jnp.where(ciota == sel, NEG, m)
    sel_all = jnp.concatenate(cols, axis=1)
    flat_ref[...] = sel_all + qiota * nchunks


def _emit_topk(sims, idxs, sim_ref, idx_ref):
    sim_ref[...] = jnp.concatenate(sims, axis=1)
    arr = jnp.concatenate(idxs[1:], axis=1)
    cols = []
    for _ in range(POSK):
        mn = jnp.min(arr, axis=1, keepdims=True)
        cols.append(mn)
        arr = jnp.where(arr == mn, IMAX, arr)
    idx_ref[...] = jnp.concatenate(cols, axis=1)


def _topk_body(nchunks, chunk, qblk, rheads, cand_ref, flat_ref,
               sim_ref, idx_ref):
    b = pl.program_id(0)
    flat = flat_ref[...]
    nsel = flat.shape[1]
    qloc = lax.broadcasted_iota(jnp.int32, (qblk, 1), 0) + b * qblk
    chunk_ids = flat - qloc * nchunks
    gidx = (chunk_ids[:, :, None] * chunk
            + lax.broadcasted_iota(jnp.int32, (qblk, nsel, chunk), 2))

    vals = cand_ref[...]
    hv, hi = [], []
    for _ in range(rheads):
        m2 = jnp.max(vals, axis=2)
        wi = jnp.where(vals == m2[:, :, None], gidx, IMAX)
        s2 = jnp.min(wi, axis=2)
        hv.append(m2[:, :, None])
        hi.append(s2[:, :, None])
        vals = jnp.where(gidx == s2[:, :, None], NEG, vals)
    heads = jnp.concatenate(hv, axis=2)
    hidx = jnp.concatenate(hi, axis=2)
    last = hidx[:, :, rheads - 1]

    sims, idxs = [], []
    exh = jnp.zeros((qblk, 1), jnp.bool_)
    hc = heads
    for _ in range(TOPK):
        m2 = jnp.max(hc, axis=2)
        mx = jnp.max(m2, axis=1, keepdims=True)
        wi = jnp.where(hc == mx[:, :, None], hidx, IMAX)
        s2 = jnp.min(wi, axis=2)
        sel = jnp.min(s2, axis=1, keepdims=True)
        sims.append(mx)
        idxs.append(sel)
        exh = exh | jnp.any(sel == last, axis=1, keepdims=True)
        hc = jnp.where(hidx == sel[:, :, None], NEG, hc)
    exhausted = jnp.any(exh)

    @pl.when(jnp.logical_not(exhausted))
    def _fast():
        _emit_topk(sims, idxs, sim_ref, idx_ref)

    @pl.when(exhausted)
    def _slow():
        vals = cand_ref[...]
        fsims, fidxs = [], []
        for _ in range(TOPK):
            m2 = jnp.max(vals, axis=2)
            mx = jnp.max(m2, axis=1, keepdims=True)
            w = jnp.where(vals == mx[:, :, None], gidx, IMAX)
            s2 = jnp.min(w, axis=2)
            sel = jnp.min(s2, axis=1, keepdims=True)
            fsims.append(mx)
            fidxs.append(sel)
            vals = jnp.where(gidx == sel[:, :, None], NEG, vals)
        _emit_topk(fsims, fidxs, sim_ref, idx_ref)


def _sc_gather(table, idx3d, rows, gchunk, row_w):
    nsub = idx3d.shape[0]
    gpw = idx3d.shape[1]
    mesh = plsc.VectorSubcoreMesh(core_axis_name="c", subcore_axis_name="s")

    @functools.partial(
        pl.kernel,
        mesh=mesh,
        out_type=jax.ShapeDtypeStruct((rows, row_w), jnp.float32),
    scratch_types=[
            pltpu.VMEM((gpw, gchunk), jnp.int32),
            pltpu.VMEM((gchunk, row_w), jnp.float32),
            pltpu.VMEM((gchunk, row_w), jnp.float32),
            pltpu.SemaphoreType.DMA,
            pltpu.SemaphoreType.DMA,
        ],
    )
    def gather_k(table_hbm, idx_hbm, out_hbm, idx_v, buf0, buf1, sem0, sem1):
        wid = lax.axis_index("s") * 2 + lax.axis_index("c")
        pltpu.sync_copy(idx_hbm.at[wid], idx_v)
        bufs, sems = (buf0, buf1), (sem0, sem1)
        handles = [None, None]
        handles[0] = pltpu.async_copy(table_hbm.at[idx_v.at[0]], buf0, sem0)
        for j in range(gpw):
            cur = j % 2
            if j + 1 < gpw:
                nxt = (j + 1) % 2
                handles[nxt] = pltpu.async_copy(
                    table_hbm.at[idx_v.at[j + 1]], bufs[nxt], sems[nxt])
            handles[cur].wait()
            pltpu.sync_copy(bufs[cur],
                            out_hbm.at[pl.ds((wid * gpw + j) * gchunk, gchunk)])

    return gather_k(table, idx3d)


def kernel(video_mem, audio_mem, query_idx):
    n, d = video_mem.shape
    q = query_idx.shape[0]
    npad = ((n + TILE - 1) // TILE) * TILE
    nchunks = npad // CHUNK
    ntiles = npad // TILE

    vq = jnp.take(video_mem, query_idx, axis=0)
    aq = jnp.take(audio_mem, query_idx, axis=0)

    sim, cmax = pl.pallas_call(
        functools.partial(_sim_body, n, TILE, CHUNK),
        grid=(ntiles,),
        in_specs=[
            pl.BlockSpec((q, d), lambda t: (0, 0)),
            pl.BlockSpec((q, d), lambda t: (0, 0)),
            pl.BlockSpec((TILE, d), lambda t: (t, 0)),
            pl.BlockSpec((TILE, d), lambda t: (t, 0)),
        ],
        out_specs=[
            pl.BlockSpec((q, TILE // CHUNK, CHUNK), lambda t: (0, t, 0)),
            pl.BlockSpec((1, q, TILE // CHUNK), lambda t: (t, 0, 0)),
        ],
        out_shape=[
            jax.ShapeDtypeStruct((q, nchunks, CHUNK), jnp.float32),
            jax.ShapeDtypeStruct((ntiles, q, TILE // CHUNK), jnp.float32),
        ],
    )(vq, aq, video_mem, audio_mem)
    cmax = jnp.transpose(cmax, (1, 0, 2)).reshape(q, nchunks)

    flat = pl.pallas_call(
        functools.partial(_chunksel_body, NSEL, nchunks),
        in_specs=[pl.BlockSpec((q, nchunks), lambda: (0, 0))],
        out_specs=pl.BlockSpec((q, NSEL), lambda: (0, 0)),
        out_shape=jax.ShapeDtypeStruct((q, NSEL), jnp.int32),
    )(cmax)

    rows = q * NSEL
    table = jnp.reshape(sim, (q * nchunks, CHUNK))
    nsub = 32
    idx3d = jnp.reshape(flat, (nsub, rows // (nsub * GCHUNK), GCHUNK))
    cand = _sc_gather(table, idx3d, rows, GCHUNK, CHUNK)
    cand = jnp.reshape(cand, (q, NSEL, CHUNK))

    qblk = 256
    pos_sim, pos_index = pl.pallas_call(
        functools.partial(_topk_body, nchunks, CHUNK, qblk, RHEADS),
        grid=(q // qblk,),
        in_specs=[
            pl.BlockSpec((qblk, NSEL, CHUNK), lambda b: (b, 0, 0)),
            pl.BlockSpec((qblk, NSEL), lambda b: (b, 0)),
        ],
        out_specs=[
            pl.BlockSpec((qblk, TOPK), lambda b: (b, 0)),
            pl.BlockSpec((qblk, POSK), lambda b: (b, 0)),
        ],
        out_shape=[
            jax.ShapeDtypeStruct((q, TOPK), jnp.float32),
            jax.ShapeDtypeStruct((q, POSK), jnp.int32),
        ],
    )(cand, flat)

    return pos_sim, pos_index


# speedup vs baseline: 1.5159x; 1.5159x over previous
import functools

import jax
import jax.numpy as jnp
from jax import lax
from jax.experimental import pallas as pl
from jax.experimental.pallas import tpu as pltpu
from jax.experimental.pallas import tpu_sc as plsc

POSK = 32
TOPK = POSK + 1
CHUNK = 128
NSEL = 36
RHEADS = 8
TILE = 1024
GCHUNK = 128
NEG = -1e30
IMAX = 2**31 - 1


def _sim_body(nvalid, tile, chunk, vq_ref, aq_ref, vt_ref, at_ref,
              sim_ref, cmax_ref):
    t = pl.program_id(0)

    def norm_rows(x):
        ss = jnp.sum(x * x, axis=1, keepdims=True)
        return x / jnp.sqrt(jnp.maximum(ss, 1e-30))

    vqn = norm_rows(vq_ref[...])
    aqn = norm_rows(aq_ref[...])
    vtn = norm_rows(vt_ref[...])
    atn = norm_rows(at_ref[...])
    dn = (((1,), (1,)), ((), ()))
    sv = lax.dot_general(vqn, vtn, dn, preferred_element_type=jnp.float32)
    sa = lax.dot_general(aqn, atn, dn, preferred_element_type=jnp.float32)
    s = jnp.minimum(sv, sa)
    nidx = t * tile + lax.broadcasted_iota(jnp.int32, (1, tile), 1)
    s = jnp.where(nidx < nvalid, s, NEG)
    q = s.shape[0]
    s3 = s.reshape(q, tile // chunk, chunk)
    sim_ref[...] = s3
    cmax_ref[...] = jnp.max(s3, axis=2)[None]


def _chunksel_body(nsel, nchunks, cmax_ref, flat_ref):
    m = cmax_ref[...]
    q = m.shape[0]
    ciota = lax.broadcasted_iota(jnp.int32, (q, nchunks), 1)
    qiota = lax.broadcasted_iota(jnp.int32, (q, 1), 0)
    cols = []
    for _ in range(nsel):
        mx = jnp.max(m, axis=1, keepdims=True)
        sel = jnp.min(jnp.where(m == mx, ciota, IMAX), axis=1, keepdims=True)
        cols.append(sel)
        m = jnp.where(ciota == sel, NEG, m)
    sel_all = jnp.concatenate(cols, axis=1)
    flat_ref[...] = sel_all + qiota * nchunks


def _emit_topk(sims, idxs, sim_ref, idx_ref):
    sim_ref[...] = jnp.concatenate(sims, axis=1)
    arr = jnp.concatenate(idxs[1:], axis=1)
    cols = []
    for _ in range(POSK):
        mn = jnp.min(arr, axis=1, keepdims=True)
        cols.append(mn)
        arr = jnp.where(arr == mn, IMAX, arr)
    idx_ref[...] = jnp.concatenate(cols, axis=1)


def _topk_body(nchunks, chunk, qblk, rheads, cand_ref, flat_ref,
               sim_ref, idx_ref):
    b = pl.program_id(0)
    flat = flat_ref[...]
    nsel = flat.shape[1]
    qloc = lax.broadcasted_iota(jnp.int32, (qblk, 1), 0) + b * qblk
    chunk_ids = flat - qloc * nchunks
    gidx = (chunk_ids[:, :, None] * chunk
            + lax.broadcasted_iota(jnp.int32, (qblk, nsel, chunk), 2))

    vals = cand_ref[...]
    hv, hi = [], []
    for _ in range(rheads):
        m2 = jnp.max(vals, axis=2)
        wi = jnp.where(vals == m2[:, :, None], gidx, IMAX)
        s2 = jnp.min(wi, axis=2)
        hv.append(m2)
        hi.append(s2)
        vals = jnp.where(gidx == s2[:, :, None], NEG, vals)
    heads = jnp.concatenate(hv, axis=1)
    hidx = jnp.concatenate(hi, axis=1)
    last = hi[rheads - 1]

    sims, idxs = [], []
    exh = jnp.zeros((qblk, 1), jnp.bool_)
    for _ in range(TOPK):
        mx = jnp.max(heads, axis=1, keepdims=True)
        wi = jnp.where(heads == mx, hidx, IMAX)
        sel = jnp.min(wi, axis=1, keepdims=True)
        sims.append(mx)
        idxs.append(sel)
        exh = exh | jnp.any(sel == last, axis=1, keepdims=True)
        heads = jnp.where(hidx == sel, NEG, heads)
    exhausted = jnp.any(exh)

    @pl.when(jnp.logical_not(exhausted))
    def _fast():
        _emit_topk(sims, idxs, sim_ref, idx_ref)

    @pl.when(exhausted)
    def _slow():
        vals = cand_ref[...]
        fsims, fidxs = [], []
        for _ in range(TOPK):
            m2 = jnp.max(vals, axis=2)
            mx = jnp.max(m2, axis=1, keepdims=True)
            w = jnp.where(vals == mx[:, :, None], gidx, IMAX)
            s2 = jnp.min(w, axis=2)
            sel = jnp.min(s2, axis=1, keepdims=True)
            fsims.append(mx)
            fidxs.append(sel)
            vals = jnp.where(gidx == sel[:, :, None], NEG, vals)
        _emit_topk(fsims, fidxs, sim_ref, idx_ref)


def _sc_gather(table, idx3d, rows, gchunk, row_w):
    nsub = idx3d.shape[0]
    gpw = idx3d.shape[1]
    mesh = plsc.VectorSubcoreMesh(core_axis_name="c", subcore_axis_name="s")

    @functools.partial(
        pl.kernel,
        mesh=mesh,
        out_type=jax.ShapeDtypeStruct((rows, row_w), jnp.float32),
    scratch_types=[
            pltpu.VMEM((gpw, gchunk), jnp.int32),
            pltpu.VMEM((gchunk, row_w), jnp.float32),
            pltpu.VMEM((gchunk, row_w), jnp.float32),
            pltpu.SemaphoreType.DMA,
            pltpu.SemaphoreType.DMA,
        ],
    )
    def gather_k(table_hbm, idx_hbm, out_hbm, idx_v, buf0, buf1, sem0, sem1):
        wid = lax.axis_index("s") * 2 + lax.axis_index("c")
        pltpu.sync_copy(idx_hbm.at[wid], idx_v)
        bufs, sems = (buf0, buf1), (sem0, sem1)
        handles = [None, None]
        handles[0] = pltpu.async_copy(table_hbm.at[idx_v.at[0]], buf0, sem0)
        for j in range(gpw):
            cur = j % 2
            if j + 1 < gpw:
                nxt = (j + 1) % 2
                handles[nxt] = pltpu.async_copy(
                    table_hbm.at[idx_v.at[j + 1]], bufs[nxt], sems[nxt])
            handles[cur].wait()
            pltpu.sync_copy(bufs[cur],
                            out_hbm.at[pl.ds((wid * gpw + j) * gchunk, gchunk)])

    return gather_k(table, idx3d)


def kernel(video_mem, audio_mem, query_idx):
    n, d = video_mem.shape
    q = query_idx.shape[0]
    npad = ((n + TILE - 1) // TILE) * TILE
    nchunks = npad // CHUNK
    ntiles = npad // TILE

    vq = jnp.take(video_mem, query_idx, axis=0)
    aq = jnp.take(audio_mem, query_idx, axis=0)

    sim, cmax = pl.pallas_call(
        functools.partial(_sim_body, n, TILE, CHUNK),
        grid=(ntiles,),
        in_specs=[
            pl.BlockSpec((q, d), lambda t: (0, 0)),
            pl.BlockSpec((q, d), lambda t: (0, 0)),
            pl.BlockSpec((TILE, d), lambda t: (t, 0)),
            pl.BlockSpec((TILE, d), lambda t: (t, 0)),
        ],
        out_specs=[
            pl.BlockSpec((q, TILE // CHUNK, CHUNK), lambda t: (0, t, 0)),
            pl.BlockSpec((1, q, TILE // CHUNK), lambda t: (t, 0, 0)),
        ],
        out_shape=[
            jax.ShapeDtypeStruct((q, nchunks, CHUNK), jnp.float32),
            jax.ShapeDtypeStruct((ntiles, q, TILE // CHUNK), jnp.float32),
        ],
    )(vq, aq, video_mem, audio_mem)
    cmax = jnp.transpose(cmax, (1, 0, 2)).reshape(q, nchunks)

    flat = pl.pallas_call(
        functools.partial(_chunksel_body, NSEL, nchunks),
        in_specs=[pl.BlockSpec((q, nchunks), lambda: (0, 0))],
        out_specs=pl.BlockSpec((q, NSEL), lambda: (0, 0)),
        out_shape=jax.ShapeDtypeStruct((q, NSEL), jnp.int32),
    )(cmax)

    rows = q * NSEL
    table = jnp.reshape(sim, (q * nchunks, CHUNK))
    nsub = 32
    idx3d = jnp.reshape(flat, (nsub, rows // (nsub * GCHUNK), GCHUNK))
    cand = _sc_gather(table, idx3d, rows, GCHUNK, CHUNK)
    cand = jnp.reshape(cand, (q, NSEL, CHUNK))

    qblk = 128
    pos_sim, pos_index = pl.pallas_call(
        functools.partial(_topk_body, nchunks, CHUNK, qblk, RHEADS),
        grid=(q // qblk,),
        in_specs=[
            pl.BlockSpec((qblk, NSEL, CHUNK), lambda b: (b, 0, 0)),
            pl.BlockSpec((qblk, NSEL), lambda b: (b, 0)),
        ],
        out_specs=[
            pl.BlockSpec((qblk, TOPK), lambda b: (b, 0)),
            pl.BlockSpec((qblk, POSK), lambda b: (b, 0)),
        ],
        out_shape=[
            jax.ShapeDtypeStruct((q, TOPK), jnp.float32),
            jax.ShapeDtypeStruct((q, POSK), jnp.int32),
        ],
    )(cand, flat)

    return pos_sim, pos_index


# speedup vs baseline: 1.5366x; 1.0136x over previous
import functools

import jax
import jax.numpy as jnp
from jax import lax
from jax.experimental import pallas as pl
from jax.experimental.pallas import tpu as pltpu
from jax.experimental.pallas import tpu_sc as plsc

POSK = 32
TOPK = POSK + 1
CHUNK = 128
NSEL = 36
RHEADS = 8
TILE = 2048
GCHUNK = 128
NEG = -1e30
IMAX = 2**31 - 1


def _sim_body(nvalid, tile, chunk, vq_ref, aq_ref, vt_ref, at_ref,
              sim_ref, cmax_ref):
    t = pl.program_id(0)

    def norm_rows(x):
        ss = jnp.sum(x * x, axis=1, keepdims=True)
        return x / jnp.sqrt(jnp.maximum(ss, 1e-30))

    vqn = norm_rows(vq_ref[...])
    aqn = norm_rows(aq_ref[...])
    vtn = norm_rows(vt_ref[...])
    atn = norm_rows(at_ref[...])
    dn = (((1,), (1,)), ((), ()))
    sv = lax.dot_general(vqn, vtn, dn, preferred_element_type=jnp.float32)
    sa = lax.dot_general(aqn, atn, dn, preferred_element_type=jnp.float32)
    s = jnp.minimum(sv, sa)
    nidx = t * tile + lax.broadcasted_iota(jnp.int32, (1, tile), 1)
    s = jnp.where(nidx < nvalid, s, NEG)
    q = s.shape[0]
    s3 = s.reshape(q, tile // chunk, chunk)
    sim_ref[...] = s3
    cmax_ref[...] = jnp.max(s3, axis=2)[None]


def _chunksel_body(nsel, nchunks, cmax_ref, flat_ref):
    m = cmax_ref[...]
    q = m.shape[0]
    ciota = lax.broadcasted_iota(jnp.int32, (q, nchunks), 1)
    qiota = lax.broadcasted_iota(jnp.int32, (q, 1), 0)
    cols = []
    for _ in range(nsel):
        mx = jnp.max(m, axis=1, keepdims=True)
        sel = jnp.min(jnp.where(m == mx, ciota, IMAX), axis=1, keepdims=True)
        cols.append(sel)
        m = jnp.where(ciota == sel, NEG, m)
    sel_all = jnp.concatenate(cols, axis=1)
    flat_ref[...] = sel_all + qiota * nchunks


def _emit_topk(sims, idxs, sim_ref, idx_ref):
    sim_ref[...] = jnp.concatenate(sims, axis=1)
    arr = jnp.concatenate(idxs[1:], axis=1)
    cols = []
    for _ in range(POSK):
        mn = jnp.min(arr, axis=1, keepdims=True)
        cols.append(mn)
        arr = jnp.where(arr == mn, IMAX, arr)
    idx_ref[...] = jnp.concatenate(cols, axis=1)


def _topk_body(nchunks, chunk, qblk, rheads, cand_ref, flat_ref,
               sim_ref, idx_ref):
    b = pl.program_id(0)
    flat = flat_ref[...]
    nsel = flat.shape[1]
    qloc = lax.broadcasted_iota(jnp.int32, (qblk, 1), 0) + b * qblk
    chunk_ids = flat - qloc * nchunks
    gidx = (chunk_ids[:, :, None] * chunk
            + lax.broadcasted_iota(jnp.int32, (qblk, nsel, chunk), 2))

    vals = cand_ref[...]
    hv, hi = [], []
    for _ in range(rheads):
        m2 = jnp.max(vals, axis=2)
        wi = jnp.where(vals == m2[:, :, None], gidx, IMAX)
        s2 = jnp.min(wi, axis=2)
        hv.append(m2)
        hi.append(s2)
        vals = jnp.where(gidx == s2[:, :, None], NEG, vals)
    heads = jnp.concatenate(hv, axis=1)
    hidx = jnp.concatenate(hi, axis=1)
    last = hi[rheads - 1]

    sims, idxs = [], []
    exh = jnp.zeros((qblk, 1), jnp.bool_)
    for _ in range(TOPK):
        mx = jnp.max(heads, axis=1, keepdims=True)
        wi = jnp.where(heads == mx, hidx, IMAX)
        sel = jnp.min(wi, axis=1, keepdims=True)
        sims.append(mx)
        idxs.append(sel)
        exh = exh | jnp.any(sel == last, axis=1, keepdims=True)
        heads = jnp.where(hidx == sel, NEG, heads)
    exhausted = jnp.any(exh)

    @pl.when(jnp.logical_not(exhausted))
    def _fast():
        _emit_topk(sims, idxs, sim_ref, idx_ref)

    @pl.when(exhausted)
    def _slow():
        vals = cand_ref[...]
        fsims, fidxs = [], []
        for _ in range(TOPK):
            m2 = jnp.max(vals, axis=2)
            mx = jnp.max(m2, axis=1, keepdims=True)
            w = jnp.where(vals == mx[:, :, None], gidx, IMAX)
            s2 = jnp.min(w, axis=2)
            sel = jnp.min(s2, axis=1, keepdims=True)
            fsims.append(mx)
            fidxs.append(sel)
            vals = jnp.where(gidx == sel[:, :, None], NEG, vals)
        _emit_topk(fsims, fidxs, sim_ref, idx_ref)


def _sc_gather(table, idx3d, rows, gchunk, row_w):
    nsub = idx3d.shape[0]
    gpw = idx3d.shape[1]
    mesh = plsc.VectorSubcoreMesh(core_axis_name="c", subcore_axis_name="s")

    @functools.partial(
        pl.kernel,
        mesh=mesh,
        out_type=jax.ShapeDtypeStruct((rows, row_w), jnp.float32),
    scratch_types=[
            pltpu.VMEM((gpw, gchunk), jnp.int32),
            pltpu.VMEM((gchunk, row_w), jnp.float32),
            pltpu.VMEM((gchunk, row_w), jnp.float32),
            pltpu.SemaphoreType.DMA,
            pltpu.SemaphoreType.DMA,
        ],
    )
    def gather_k(table_hbm, idx_hbm, out_hbm, idx_v, buf0, buf1, sem0, sem1):
        wid = lax.axis_index("s") * 2 + lax.axis_index("c")
        pltpu.sync_copy(idx_hbm.at[wid], idx_v)
        bufs, sems = (buf0, buf1), (sem0, sem1)
        handles = [None, None]
        handles[0] = pltpu.async_copy(table_hbm.at[idx_v.at[0]], buf0, sem0)
        for j in range(gpw):
            cur = j % 2
            if j + 1 < gpw:
                nxt = (j + 1) % 2
                handles[nxt] = pltpu.async_copy(
                    table_hbm.at[idx_v.at[j + 1]], bufs[nxt], sems[nxt])
            handles[cur].wait()
            pltpu.sync_copy(bufs[cur],
                            out_hbm.at[pl.ds((wid * gpw + j) * gchunk, gchunk)])

    return gather_k(table, idx3d)


def kernel(video_mem, audio_mem, query_idx):
    n, d = video_mem.shape
    q = query_idx.shape[0]
    npad = ((n + TILE - 1) // TILE) * TILE
    nchunks = npad // CHUNK
    ntiles = npad // TILE

    vq = jnp.take(video_mem, query_idx, axis=0)
    aq = jnp.take(audio_mem, query_idx, axis=0)

    sim, cmax = pl.pallas_call(
        functools.partial(_sim_body, n, TILE, CHUNK),
        grid=(ntiles,),
        in_specs=[
            pl.BlockSpec((q, d), lambda t: (0, 0)),
            pl.BlockSpec((q, d), lambda t: (0, 0)),
            pl.BlockSpec((TILE, d), lambda t: (t, 0)),
            pl.BlockSpec((TILE, d), lambda t: (t, 0)),
        ],
        out_specs=[
            pl.BlockSpec((q, TILE // CHUNK, CHUNK), lambda t: (0, t, 0)),
            pl.BlockSpec((1, q, TILE // CHUNK), lambda t: (t, 0, 0)),
        ],
        out_shape=[
            jax.ShapeDtypeStruct((q, nchunks, CHUNK), jnp.float32),
            jax.ShapeDtypeStruct((ntiles, q, TILE // CHUNK), jnp.float32),
        ],
    )(vq, aq, video_mem, audio_mem)
    cmax = jnp.transpose(cmax, (1, 0, 2)).reshape(q, nchunks)

    flat = pl.pallas_call(
        functools.partial(_chunksel_body, NSEL, nchunks),
        in_specs=[pl.BlockSpec((q, nchunks), lambda: (0, 0))],
        out_specs=pl.BlockSpec((q, NSEL), lambda: (0, 0)),
        out_shape=jax.ShapeDtypeStruct((q, NSEL), jnp.int32),
    )(cmax)

    rows = q * NSEL
    table = jnp.reshape(sim, (q * nchunks, CHUNK))
    nsub = 32
    idx3d = jnp.reshape(flat, (nsub, rows // (nsub * GCHUNK), GCHUNK))
    cand = _sc_gather(table, idx3d, rows, GCHUNK, CHUNK)
    cand = jnp.reshape(cand, (q, NSEL, CHUNK))

    qblk = 128
    pos_sim, pos_index = pl.pallas_call(
        functools.partial(_topk_body, nchunks, CHUNK, qblk, RHEADS),
        grid=(q // qblk,),
        in_specs=[
            pl.BlockSpec((qblk, NSEL, CHUNK), lambda b: (b, 0, 0)),
            pl.BlockSpec((qblk, NSEL), lambda b: (b, 0)),
        ],
        out_specs=[
            pl.BlockSpec((qblk, TOPK), lambda b: (b, 0)),
            pl.BlockSpec((qblk, POSK), lambda b: (b, 0)),
        ],
        out_shape=[
            jax.ShapeDtypeStruct((q, TOPK), jnp.float32),
            jax.ShapeDtypeStruct((q, POSK), jnp.int32),
        ],
    )(cand, flat)

    return pos_sim, pos_index


# speedup vs baseline: 1.5616x; 1.0163x over previous
import functools

import jax
import jax.numpy as jnp
from jax import lax
from jax.experimental import pallas as pl
from jax.experimental.pallas import tpu as pltpu
from jax.experimental.pallas import tpu_sc as plsc

POSK = 32
TOPK = POSK + 1
CHUNK = 128
NSEL = 36
RHEADS = 8
TILE = 2048
GCHUNK = 128
NEG = -1e30
IMAX = 2**31 - 1


def _sim_body(nvalid, tile, chunk, vq_ref, aq_ref, vt_ref, at_ref,
              sim_ref, cmax_ref, vqn_ref, aqn_ref):
    t = pl.program_id(0)

    def norm_rows(x):
        ss = jnp.sum(x * x, axis=1, keepdims=True)
        return x / jnp.sqrt(jnp.maximum(ss, 1e-30))

    @pl.when(t == 0)
    def _norm_queries():
        vqn_ref[...] = norm_rows(vq_ref[...])
        aqn_ref[...] = norm_rows(aq_ref[...])

    vqn = vqn_ref[...]
    aqn = aqn_ref[...]
    vtn = norm_rows(vt_ref[...])
    atn = norm_rows(at_ref[...])
    dn = (((1,), (1,)), ((), ()))
    sv = lax.dot_general(vqn, vtn, dn, preferred_element_type=jnp.float32)
    sa = lax.dot_general(aqn, atn, dn, preferred_element_type=jnp.float32)
    s = jnp.minimum(sv, sa)
    nidx = t * tile + lax.broadcasted_iota(jnp.int32, (1, tile), 1)
    s = jnp.where(nidx < nvalid, s, NEG)
    q = s.shape[0]
    s3 = s.reshape(q, tile // chunk, chunk)
    sim_ref[...] = s3
    cmax_ref[...] = jnp.max(s3, axis=2)[None]


def _chunksel_body(nsel, nchunks, cmax_ref, flat_ref):
    m = cmax_ref[...]
    q = m.shape[0]
    ciota = lax.broadcasted_iota(jnp.int32, (q, nchunks), 1)
    qiota = lax.broadcasted_iota(jnp.int32, (q, 1), 0)
    cols = []
    for _ in range(nsel):
        mx = jnp.max(m, axis=1, keepdims=True)
        sel = jnp.min(jnp.where(m == mx, ciota, IMAX), axis=1, keepdims=True)
        cols.append(sel)
        m = jnp.where(ciota == sel, NEG, m)
    sel_all = jnp.concatenate(cols, axis=1)
    flat_ref[...] = sel_all + qiota * nchunks


def _emit_topk(sims, idxs, sim_ref, idx_ref):
    sim_ref[...] = jnp.concatenate(sims, axis=1)
    arr = jnp.concatenate(idxs[1:], axis=1)
    cols = []
    for _ in range(POSK):
        mn = jnp.min(arr, axis=1, keepdims=True)
        cols.append(mn)
        arr = jnp.where(arr == mn, IMAX, arr)
    idx_ref[...] = jnp.concatenate(cols, axis=1)


def _topk_body(nchunks, chunk, qblk, rheads, cand_ref, flat_ref,
               sim_ref, idx_ref):
    b = pl.program_id(0)
    flat = flat_ref[...]
    nsel = flat.shape[1]
    qloc = lax.broadcasted_iota(jnp.int32, (qblk, 1), 0) + b * qblk
    chunk_ids = flat - qloc * nchunks
    gidx = (chunk_ids[:, :, None] * chunk
            + lax.broadcasted_iota(jnp.int32, (qblk, nsel, chunk), 2))

    vals = cand_ref[...]
    hv, hi = [], []
    for _ in range(rheads):
        m2 = jnp.max(vals, axis=2)
        wi = jnp.where(vals == m2[:, :, None], gidx, IMAX)
        s2 = jnp.min(wi, axis=2)
        hv.append(m2)
        hi.append(s2)
        vals = jnp.where(gidx == s2[:, :, None], NEG, vals)
    heads = jnp.concatenate(hv, axis=1)
    hidx = jnp.concatenate(hi, axis=1)
    last = hi[rheads - 1]

    sims, idxs = [], []
    exh = jnp.zeros((qblk, 1), jnp.bool_)
    for _ in range(TOPK):
        mx = jnp.max(heads, axis=1, keepdims=True)
        wi = jnp.where(heads == mx, hidx, IMAX)
        sel = jnp.min(wi, axis=1, keepdims=True)
        sims.append(mx)
        idxs.append(sel)
        exh = exh | jnp.any(sel == last, axis=1, keepdims=True)
        heads = jnp.where(hidx == sel, NEG, heads)
    exhausted = jnp.any(exh)

    @pl.when(jnp.logical_not(exhausted))
    def _fast():
        _emit_topk(sims, idxs, sim_ref, idx_ref)

    @pl.when(exhausted)
    def _slow():
        vals = cand_ref[...]
        fsims, fidxs = [], []
        for _ in range(TOPK):
            m2 = jnp.max(vals, axis=2)
            mx = jnp.max(m2, axis=1, keepdims=True)
            w = jnp.where(vals == mx[:, :, None], gidx, IMAX)
            s2 = jnp.min(w, axis=2)
            sel = jnp.min(s2, axis=1, keepdims=True)
            fsims.append(mx)
            fidxs.append(sel)
            vals = jnp.where(gidx == sel[:, :, None], NEG, vals)
        _emit_topk(fsims, fidxs, sim_ref, idx_ref)


def _sc_gather(table, idx3d, rows, gchunk, row_w):
    nsub = idx3d.shape[0]
    gpw = idx3d.shape[1]
    mesh = plsc.VectorSubcoreMesh(core_axis_name="c", subcore_axis_name="s")

    @functools.partial(
        pl.kernel,
        mesh=mesh,
        out_type=jax.ShapeDtypeStruct((rows, row_w), jnp.float32),
    scratch_types=[
            pltpu.VMEM((gpw, gchunk), jnp.int32),
            pltpu.VMEM((gchunk, row_w), jnp.float32),
            pltpu.VMEM((gchunk, row_w), jnp.float32),
            pltpu.SemaphoreType.DMA,
            pltpu.SemaphoreType.DMA,
        ],
    )
    def gather_k(table_hbm, idx_hbm, out_hbm, idx_v, buf0, buf1, sem0, sem1):
        wid = lax.axis_index("s") * 2 + lax.axis_index("c")
        pltpu.sync_copy(idx_hbm.at[wid], idx_v)
        bufs, sems = (buf0, buf1), (sem0, sem1)
        handles = [None, None]
        handles[0] = pltpu.async_copy(table_hbm.at[idx_v.at[0]], buf0, sem0)
        for j in range(gpw):
            cur = j % 2
            if j + 1 < gpw:
                nxt = (j + 1) % 2
                handles[nxt] = pltpu.async_copy(
                    table_hbm.at[idx_v.at[j + 1]], bufs[nxt], sems[nxt])
            handles[cur].wait()
            pltpu.sync_copy(bufs[cur],
                            out_hbm.at[pl.ds((wid * gpw + j) * gchunk, gchunk)])

    return gather_k(table, idx3d)


def kernel(video_mem, audio_mem, query_idx):
    n, d = video_mem.shape
    q = query_idx.shape[0]
    npad = ((n + TILE - 1) // TILE) * TILE
    nchunks = npad // CHUNK
    ntiles = npad // TILE

    vq = jnp.take(video_mem, query_idx, axis=0)
    aq = jnp.take(audio_mem, query_idx, axis=0)

    sim, cmax = pl.pallas_call(
        functools.partial(_sim_body, n, TILE, CHUNK),
        grid=(ntiles,),
        in_specs=[
            pl.BlockSpec((q, d), lambda t: (0, 0)),
            pl.BlockSpec((q, d), lambda t: (0, 0)),
            pl.BlockSpec((TILE, d), lambda t: (t, 0)),
            pl.BlockSpec((TILE, d), lambda t: (t, 0)),
        ],
        out_specs=[
            pl.BlockSpec((q, TILE // CHUNK, CHUNK), lambda t: (0, t, 0)),
            pl.BlockSpec((1, q, TILE // CHUNK), lambda t: (t, 0, 0)),
        ],
        out_shape=[
            jax.ShapeDtypeStruct((q, nchunks, CHUNK), jnp.float32),
            jax.ShapeDtypeStruct((ntiles, q, TILE // CHUNK), jnp.float32),
        ],
        scratch_shapes=[
            pltpu.VMEM((q, d), jnp.float32),
            pltpu.VMEM((q, d), jnp.float32),
        ],
    )(vq, aq, video_mem, audio_mem)
    cmax = jnp.transpose(cmax, (1, 0, 2)).reshape(q, nchunks)

    flat = pl.pallas_call(
        functools.partial(_chunksel_body, NSEL, nchunks),
        in_specs=[pl.BlockSpec((q, nchunks), lambda: (0, 0))],
        out_specs=pl.BlockSpec((q, NSEL), lambda: (0, 0)),
        out_shape=jax.ShapeDtypeStruct((q, NSEL), jnp.int32),
    )(cmax)

    rows = q * NSEL
    table = jnp.reshape(sim, (q * nchunks, CHUNK))
    nsub = 32
    idx3d = jnp.reshape(flat, (nsub, rows // (nsub * GCHUNK), GCHUNK))
    cand = _sc_gather(table, idx3d, rows, GCHUNK, CHUNK)
    cand = jnp.reshape(cand, (q, NSEL, CHUNK))

    qblk = 128
    pos_sim, pos_index = pl.pallas_call(
        functools.partial(_topk_body, nchunks, CHUNK, qblk, RHEADS),
        grid=(q // qblk,),
        in_specs=[
            pl.BlockSpec((qblk, NSEL, CHUNK), lambda b: (b, 0, 0)),
            pl.BlockSpec((qblk, NSEL), lambda b: (b, 0)),
        ],
        out_specs=[
            pl.BlockSpec((qblk, TOPK), lambda b: (b, 0)),
            pl.BlockSpec((qblk, POSK), lambda b: (b, 0)),
        ],
        out_shape=[
            jax.ShapeDtypeStruct((q, TOPK), jnp.float32),
            jax.ShapeDtypeStruct((q, POSK), jnp.int32),
        ],
    )(cand, flat)

    return pos_sim, pos_index
